# baseline (speedup 1.0000x reference)
import functools

import jax
import jax.numpy as jnp
from jax import lax
from jax.experimental import pallas as pl
from jax.experimental.pallas import tpu as pltpu
from jax.experimental.pallas import tpu_sc as plsc

N = 10000
E = 320000
FEAT = 128
NCLS = 64

NC = 2
NS = 16
NW = NC * NS
NPAD = 10240
NPS = NPAD // NS
K = 128
CPT = 80
EPT = CPT * K
EPAD = NW * EPT
NB = 2
HCPT = CPT // 2
GB = 8

_MESH = plsc.VectorSubcoreMesh(core_axis_name="c", subcore_axis_name="s")


@functools.partial(
    pl.kernel,
    out_type=jax.ShapeDtypeStruct((NC, NPAD, FEAT), jnp.float32),
    mesh=_MESH,
    scratch_types=[
        pltpu.VMEM((CPT, K), jnp.int32),
        pltpu.VMEM((K, FEAT), jnp.float32),
        pltpu.VMEM_SHARED((NPAD, FEAT), jnp.float32),
        pltpu.SemaphoreType.DMA,
    ],
)
def _deg_kernel(dst_hbm, ones_hbm, zeros_hbm, out_hbm, didx, ones_v, hist, sem):
    c = lax.axis_index("c")
    s = lax.axis_index("s")
    wid = c * NS + s
    pltpu.sync_copy(ones_hbm, ones_v)
    pltpu.sync_copy(dst_hbm.at[wid], didx)
    pltpu.sync_copy(zeros_hbm, hist.at[pl.ds(s * NPS, NPS)])
    plsc.subcore_barrier()

    def body(g, carry):
        for b in range(GB):
            pltpu.async_copy(ones_v, hist.at[didx.at[g * GB + b]], sem,
                             add=True)
        for b in range(GB):
            pltpu.make_async_copy(ones_v, hist.at[didx.at[g * GB + b]],
                                  sem).wait()
        return carry

    lax.fori_loop(0, CPT // GB, body, 0)
    plsc.subcore_barrier()
    pltpu.sync_copy(hist.at[pl.ds(s * NPS, NPS)],
                    out_hbm.at[c, pl.ds(s * NPS, NPS)])


def _make_prop(feat):
    @functools.partial(
        pl.kernel,
        out_type=jax.ShapeDtypeStruct((NC, NPAD, feat), jnp.float32),
        mesh=_MESH,
        scratch_types=[
            pltpu.VMEM((HCPT, K), jnp.int32),
            pltpu.VMEM((HCPT, K), jnp.int32),
            pltpu.VMEM((NB, K, feat), jnp.float32),
            pltpu.VMEM_SHARED((NPAD, feat), jnp.float32),
            pltpu.SemaphoreType.DMA,
        ],
    )
    def prop(y_hbm, src_hbm, dst_hbm, zeros_hbm, out_hbm,
             sidx, didx, rows, acc, gsem):
        c = lax.axis_index("c")
        s = lax.axis_index("s")
        wid = c * NS + s
        pltpu.sync_copy(zeros_hbm, acc.at[pl.ds(s * NPS, NPS)])
        plsc.subcore_barrier()

        for h in range(2):
            pltpu.sync_copy(src_hbm.at[wid, pl.ds(h * HCPT, HCPT)], sidx)
            pltpu.sync_copy(dst_hbm.at[wid, pl.ds(h * HCPT, HCPT)], didx)
            for b in range(NB):
                pltpu.async_copy(y_hbm.at[sidx.at[b]], rows.at[b], gsem)

            def body(g, carry):
                for b in range(NB):
                    j = g * NB + b
                    pltpu.make_async_copy(y_hbm.at[pl.ds(0, K)], rows.at[b],
                                          gsem).wait()
                    pltpu.sync_copy(rows.at[b], acc.at[didx.at[j]], add=True)
                    pltpu.async_copy(y_hbm.at[sidx.at[j + NB]], rows.at[b],
                                     gsem)
                return carry

            lax.fori_loop(0, (HCPT - NB) // NB, body, 0)
            for b in range(NB):
                j = HCPT - NB + b
                pltpu.make_async_copy(y_hbm.at[pl.ds(0, K)], rows.at[b],
                                      gsem).wait()
                pltpu.sync_copy(rows.at[b], acc.at[didx.at[j]], add=True)

        plsc.subcore_barrier()
        pltpu.sync_copy(acc.at[pl.ds(s * NPS, NPS)],
                        out_hbm.at[c, pl.ds(s * NPS, NPS)])

    return prop


_prop128 = _make_prop(FEAT)


_BRP = 1024
_BR = 1000


def _tc_first(x, W1, cnt):
    def body(x_ref, w_ref, c_ref, y_ref, dv_ref):
        h = jnp.dot(x_ref[...], w_ref[...], preferred_element_type=jnp.float32)
        deg = c_ref[0, :, 0] + c_ref[1, :, 0] + 1.0
        dv = lax.rsqrt(deg)
        y_ref[...] = h * dv[:, None]
        dv_ref[...] = dv[:, None]

    return pl.pallas_call(
        body,
        grid=(N // _BR,),
        in_specs=[
            pl.BlockSpec((_BR, FEAT), lambda i: (i, 0)),
            pl.BlockSpec((FEAT, FEAT), lambda i: (0, 0)),
            pl.BlockSpec((NC, _BR, FEAT), lambda i: (0, i, 0)),
        ],
        out_specs=[
            pl.BlockSpec((_BR, FEAT), lambda i: (i, 0)),
            pl.BlockSpec((_BR, 1), lambda i: (i, 0)),
        ],
        out_shape=[
            jax.ShapeDtypeStruct((NPAD, FEAT), jnp.float32),
            jax.ShapeDtypeStruct((NPAD, 1), jnp.float32),
        ],
    )(x, W1, cnt)


def _tc_mid(p, y, dinv, b, W):
    def body(p_ref, y_ref, dv_ref, b_ref, w_ref, o_ref):
        t = dv_ref[...] * (p_ref[0] + p_ref[1] + y_ref[...]) + b_ref[...]
        a = jnp.tanh(t)
        o_ref[...] = dv_ref[...] * jnp.dot(
            a, w_ref[...], preferred_element_type=jnp.float32)

    return pl.pallas_call(
        body,
        grid=(NPAD // _BRP,),
        in_specs=[
            pl.BlockSpec((NC, _BRP, FEAT), lambda i: (0, i, 0)),
            pl.BlockSpec((_BRP, FEAT), lambda i: (i, 0)),
            pl.BlockSpec((_BRP, 1), lambda i: (i, 0)),
            pl.BlockSpec((1, FEAT), lambda i: (0, 0)),
            pl.BlockSpec((FEAT, FEAT), lambda i: (0, 0)),
        ],
        out_specs=pl.BlockSpec((_BRP, FEAT), lambda i: (i, 0)),
        out_shape=jax.ShapeDtypeStruct((NPAD, FEAT), jnp.float32),
    )(p, y, dinv, b, W)


def _tc_last(p, y, dinv, b):
    def body(p_ref, y_ref, dv_ref, b_ref, o_ref):
        t = dv_ref[...] * (p_ref[0] + p_ref[1] + y_ref[...])
        o_ref[...] = t[:, :NCLS] + b_ref[...]

    return pl.pallas_call(
        body,
        grid=(N // _BR,),
        in_specs=[
            pl.BlockSpec((NC, _BR, FEAT), lambda i: (0, i, 0)),
            pl.BlockSpec((_BR, FEAT), lambda i: (i, 0)),
            pl.BlockSpec((_BR, 1), lambda i: (i, 0)),
            pl.BlockSpec((1, NCLS), lambda i: (0, 0)),
        ],
        out_specs=pl.BlockSpec((_BR, NCLS), lambda i: (i, 0)),
        out_shape=jax.ShapeDtypeStruct((N, NCLS), jnp.float32),
    )(p, y, dinv, b)


def kernel(x, adj, W1, b1, W2, b2, W3, b3):
    adj = adj.astype(jnp.int32)
    fill = N + jnp.arange(EPAD - E, dtype=jnp.int32) % (NPAD - N)
    src = jnp.concatenate([adj[0], fill]).reshape(NW, CPT, K)
    dst = jnp.concatenate([adj[1], fill]).reshape(NW, CPT, K)
    ones128 = jnp.ones((K, FEAT), jnp.float32)
    zeros128 = jnp.zeros((NPS, FEAT), jnp.float32)
    W3p = jnp.pad(W3, ((0, 0), (0, FEAT - NCLS)))

    cnt = _deg_kernel(dst, ones128, zeros128)
    y1, dinv = _tc_first(x, W1, cnt)
    p = _prop128(y1, src, dst, zeros128)
    y2 = _tc_mid(p, y1, dinv, b1.reshape(1, -1), W2)
    p = _prop128(y2, src, dst, zeros128)
    y3 = _tc_mid(p, y2, dinv, b2.reshape(1, -1), W3p)
    p = _prop128(y3, src, dst, zeros128)
    return _tc_last(p, y3, dinv, b3.reshape(1, -1))


# speedup vs baseline: 1.0239x; 1.0239x over previous
import functools

import jax
import jax.numpy as jnp
from jax import lax
from jax.experimental import pallas as pl
from jax.experimental.pallas import tpu as pltpu
from jax.experimental.pallas import tpu_sc as plsc

N = 10000
E = 320000
FEAT = 128
NCLS = 64

NC = 2
NS = 16
NW = NC * NS
NPAD = 10240
NPS = NPAD // NS
K = 128
CPT = 80
EPT = CPT * K
EPAD = NW * EPT
NB = 2
RB = 8
GB = 8

_MESH = plsc.VectorSubcoreMesh(core_axis_name="c", subcore_axis_name="s")


@functools.partial(
    pl.kernel,
    out_type=jax.ShapeDtypeStruct((NC, NPAD, FEAT), jnp.float32),
    mesh=_MESH,
    scratch_types=[
        pltpu.VMEM((CPT, K), jnp.int32),
        pltpu.VMEM((K, FEAT), jnp.float32),
        pltpu.VMEM_SHARED((NPAD, FEAT), jnp.float32),
        pltpu.SemaphoreType.DMA,
    ],
)
def _deg_kernel(dst_hbm, ones_hbm, zeros_hbm, out_hbm, didx, ones_v, hist, sem):
    c = lax.axis_index("c")
    s = lax.axis_index("s")
    wid = c * NS + s
    pltpu.sync_copy(ones_hbm, ones_v)
    pltpu.sync_copy(dst_hbm.at[wid], didx)
    pltpu.sync_copy(zeros_hbm.at[pl.ds(s * NPS, NPS)],
                    hist.at[pl.ds(s * NPS, NPS)])
    plsc.subcore_barrier()

    def body(g, carry):
        for b in range(GB):
            pltpu.async_copy(ones_v, hist.at[didx.at[g * GB + b]], sem,
                             add=True)
        for b in range(GB):
            pltpu.make_async_copy(ones_v, hist.at[didx.at[g * GB + b]],
                                  sem).wait()
        return carry

    lax.fori_loop(0, CPT // GB, body, 0)
    plsc.subcore_barrier()
    pltpu.sync_copy(hist.at[pl.ds(s * NPS, NPS)],
                    out_hbm.at[c, pl.ds(s * NPS, NPS)])


def _make_prop(feat):
    @functools.partial(
        pl.kernel,
        out_type=jax.ShapeDtypeStruct((NC, NPAD, feat), jnp.float32),
        mesh=_MESH,
        scratch_types=[
            pltpu.VMEM((RB, K), jnp.int32),
            pltpu.VMEM((RB, K), jnp.int32),
            pltpu.VMEM((NB, K, feat), jnp.float32),
            pltpu.VMEM_SHARED((NPAD, feat), jnp.float32),
            pltpu.SemaphoreType.DMA,
            pltpu.SemaphoreType.DMA,
        ],
    )
    def prop(y_hbm, src_hbm, dst_hbm, zeros_hbm, out_hbm,
             sidx, didx, rows, acc, gsem, isem):
        c = lax.axis_index("c")
        s = lax.axis_index("s")
        wid = c * NS + s
        pltpu.sync_copy(zeros_hbm.at[pl.ds(s * NPS, NPS)],
                        acc.at[pl.ds(s * NPS, NPS)])

        for r in range(RB):
            pltpu.async_copy(src_hbm.at[wid, r], sidx.at[r], isem)
            pltpu.async_copy(dst_hbm.at[wid, r], didx.at[r], isem)
        plsc.subcore_barrier()
        for b in range(NB):
            pltpu.make_async_copy(src_hbm.at[wid, b], sidx.at[b], isem).wait()
            pltpu.make_async_copy(dst_hbm.at[wid, b], didx.at[b], isem).wait()
            pltpu.async_copy(y_hbm.at[sidx.at[b]], rows.at[b], gsem)

        def body(g, carry):
            for b in range(NB):
                j = g * NB + b
                r = lax.rem(j, RB)
                rn = lax.rem(j + NB, RB)
                pltpu.make_async_copy(y_hbm.at[pl.ds(0, K)], rows.at[b],
                                      gsem).wait()
                pltpu.sync_copy(rows.at[b], acc.at[didx.at[r]], add=True)
                jj = jnp.minimum(j + RB, CPT - 1)
                pltpu.async_copy(src_hbm.at[wid, jj], sidx.at[r], isem)
                pltpu.async_copy(dst_hbm.at[wid, jj], didx.at[r], isem)
                pltpu.make_async_copy(src_hbm.at[wid, 0], sidx.at[0],
                                      isem).wait()
                pltpu.make_async_copy(dst_hbm.at[wid, 0], didx.at[0],
                                      isem).wait()
                pltpu.async_copy(y_hbm.at[sidx.at[rn]], rows.at[b], gsem)
            return carry

        lax.fori_loop(0, (CPT - NB) // NB, body, 0)
        for b in range(NB):
            j = CPT - NB + b
            pltpu.make_async_copy(y_hbm.at[pl.ds(0, K)], rows.at[b],
                                  gsem).wait()
            pltpu.sync_copy(rows.at[b], acc.at[didx.at[j % RB]], add=True)
        for _ in range(RB - NB):
            pltpu.make_async_copy(src_hbm.at[wid, 0], sidx.at[0], isem).wait()
            pltpu.make_async_copy(dst_hbm.at[wid, 0], didx.at[0], isem).wait()

        plsc.subcore_barrier()
        pltpu.sync_copy(acc.at[pl.ds(s * NPS, NPS)],
                        out_hbm.at[c, pl.ds(s * NPS, NPS)])

    return prop


_prop128 = _make_prop(FEAT)


_BRP = 1024
_BR = 1000


def _tc_first(x, W1, cnt):
    def body(x_ref, w_ref, c_ref, y_ref, dv_ref):
        h = jnp.dot(x_ref[...], w_ref[...], preferred_element_type=jnp.float32)
        deg = c_ref[0, :, 0] + c_ref[1, :, 0] + 1.0
        dv = lax.rsqrt(deg)
        y_ref[...] = h * dv[:, None]
        dv_ref[...] = dv[:, None]

    return pl.pallas_call(
        body,
        grid=(NPAD // _BRP,),
        in_specs=[
            pl.BlockSpec((_BRP, FEAT), lambda i: (i, 0)),
            pl.BlockSpec((FEAT, FEAT), lambda i: (0, 0)),
            pl.BlockSpec((NC, _BRP, FEAT), lambda i: (0, i, 0)),
        ],
        out_specs=[
            pl.BlockSpec((_BRP, FEAT), lambda i: (i, 0)),
            pl.BlockSpec((_BRP, 1), lambda i: (i, 0)),
        ],
        out_shape=[
            jax.ShapeDtypeStruct((NPAD, FEAT), jnp.float32),
            jax.ShapeDtypeStruct((NPAD, 1), jnp.float32),
        ],
    )(x, W1, cnt)


def _tc_mid(p, y, dinv, b, W):
    def body(p_ref, y_ref, dv_ref, b_ref, w_ref, o_ref):
        t = dv_ref[...] * (p_ref[0] + p_ref[1] + y_ref[...]) + b_ref[...]
        a = jnp.tanh(t)
        o_ref[...] = dv_ref[...] * jnp.dot(
            a, w_ref[...], preferred_element_type=jnp.float32)

    return pl.pallas_call(
        body,
        grid=(NPAD // _BRP,),
        in_specs=[
            pl.BlockSpec((NC, _BRP, FEAT), lambda i: (0, i, 0)),
            pl.BlockSpec((_BRP, FEAT), lambda i: (i, 0)),
            pl.BlockSpec((_BRP, 1), lambda i: (i, 0)),
            pl.BlockSpec((1, FEAT), lambda i: (0, 0)),
            pl.BlockSpec((FEAT, FEAT), lambda i: (0, 0)),
        ],
        out_specs=pl.BlockSpec((_BRP, FEAT), lambda i: (i, 0)),
        out_shape=jax.ShapeDtypeStruct((NPAD, FEAT), jnp.float32),
    )(p, y, dinv, b, W)


def _tc_last(p, y, dinv, b):
    def body(p_ref, y_ref, dv_ref, b_ref, o_ref):
        t = dv_ref[...] * (p_ref[0] + p_ref[1] + y_ref[...])
        o_ref[...] = t[:, :NCLS] + b_ref[...]

    return pl.pallas_call(
        body,
        grid=(N // _BR,),
        in_specs=[
            pl.BlockSpec((NC, _BR, FEAT), lambda i: (0, i, 0)),
            pl.BlockSpec((_BR, FEAT), lambda i: (i, 0)),
            pl.BlockSpec((_BR, 1), lambda i: (i, 0)),
            pl.BlockSpec((1, NCLS), lambda i: (0, 0)),
        ],
        out_specs=pl.BlockSpec((_BR, NCLS), lambda i: (i, 0)),
        out_shape=jax.ShapeDtypeStruct((N, NCLS), jnp.float32),
    )(p, y, dinv, b)


def kernel(x, adj, W1, b1, W2, b2, W3, b3):
    adj = adj.astype(jnp.int32)
    fill = N + jnp.arange(EPAD - E, dtype=jnp.int32) % (NPAD - N)
    src = jnp.concatenate([adj[0], fill]).reshape(NW, CPT, K)
    dst = jnp.concatenate([adj[1], fill]).reshape(NW, CPT, K)
    x_pad = jnp.pad(x, ((0, NPAD - N), (0, 0)))
    ones128 = jnp.ones((K, FEAT), jnp.float32)
    zeros128 = jnp.zeros((NPAD, FEAT), jnp.float32)
    W3p = jnp.pad(W3, ((0, 0), (0, FEAT - NCLS)))

    cnt = _deg_kernel(dst, ones128, zeros128)
    y1, dinv = _tc_first(x_pad, W1, cnt)
    p = _prop128(y1, src, dst, zeros128)
    y2 = _tc_mid(p, y1, dinv, b1.reshape(1, -1), W2)
    p = _prop128(y2, src, dst, zeros128)
    y3 = _tc_mid(p, y2, dinv, b2.reshape(1, -1), W3p)
    p = _prop128(y3, src, dst, zeros128)
    return _tc_last(p, y3, dinv, b3.reshape(1, -1))
